# baseline (speedup 1.0000x reference)
import jax
import jax.numpy as jnp
from jax.experimental import pallas as pl
from jax.experimental.pallas import tpu as pltpu

_NT = 4
_NT2 = 8


def _make_mlp_kernel(grid_m, tm):
    def _mlp_kernel(x_ref, w1_hbm, b1_ref, w2_hbm, b2_ref, o_ref,
                    w1s, w2s, w2stg, hring, sem1, sem2):
        I, H = w1s.shape
        O = w2s.shape[1]
        r1 = I // _NT
        r2 = H // _NT2
        s = pl.program_id(0)

        def c1(t):
            return pltpu.make_async_copy(
                w1_hbm.at[pl.ds(t * r1, r1), :],
                w1s.at[pl.ds(t * r1, r1), :], sem1.at[t])

        def c2(t):
            return pltpu.make_async_copy(
                w2_hbm.at[pl.ds(t * r2, r2), :], w2stg.at[t % 2], sem2.at[t])

        @pl.when(s == 0)
        def _start_weight_dmas():
            for t in range(_NT):
                c1(t).start()
            c2(0).start()
            c2(1).start()

        @pl.when(s == 0)
        def _dot1_gated():
            x = x_ref[...]
            h = b1_ref[...] * jnp.ones((x.shape[0], 1), jnp.float32)
            for t in range(_NT):
                c1(t).wait()
                h = h + jnp.dot(x[:, t * r1:(t + 1) * r1],
                                w1s[pl.ds(t * r1, r1), :],
                                preferred_element_type=jnp.float32)
            hring[pl.ds(0, tm), :] = jnp.maximum(h, 0.0).astype(jnp.bfloat16)

        if grid_m > 1:
            @pl.when(jnp.logical_and(s > 0, s < grid_m))
            def _dot1_steady():
                x = x_ref[...]
                base = (s % 2) * tm
                r2c = H // _NT
                for t in range(_NT):
                    cs = t * r2c
                    ht = jnp.dot(x, w1s[:, cs:cs + r2c],
                                 preferred_element_type=jnp.float32)
                    ht = jnp.maximum(ht + b1_ref[:, cs:cs + r2c], 0.0)
                    hring[pl.ds(base, tm), cs:cs + r2c] = ht.astype(jnp.bfloat16)

        @pl.when(s == 1)
        def _dot2_gated():
            hp = hring[pl.ds(0, tm), :]
            acc = b2_ref[...] * jnp.ones((tm, 1), jnp.float32)
            for t in range(_NT2):
                c2(t).wait()
                w2s[pl.ds(t * r2, r2), :] = w2stg[t % 2].astype(jnp.bfloat16)
                if t + 2 < _NT2:
                    c2(t + 2).start()
                acc = acc + jnp.dot(hp[:, t * r2:(t + 1) * r2],
                                    w2s[pl.ds(t * r2, r2), :],
                                    preferred_element_type=jnp.float32)
            o_ref[...] = acc.astype(o_ref.dtype)

        if grid_m > 1:
            @pl.when(s > 1)
            def _dot2_steady():
                base = ((s - 1) % 2) * tm
                hp = hring[pl.ds(base, tm), :]
                out = jnp.dot(hp, w2s[...],
                              preferred_element_type=jnp.float32) + b2_ref[...]
                o_ref[...] = out.astype(o_ref.dtype)

    return _mlp_kernel


def kernel(x, w1, b1, w2, b2):
    I = x.shape[-1]
    H = w1.shape[1]
    O = w2.shape[1]
    lead_shape = x.shape[:-1]

    x2 = x.reshape(-1, I)
    M = x2.shape[0]

    if M <= 512:
        tm = M
    else:
        tm = 512
    grid_m = pl.cdiv(M, tm)

    b1r = b1.reshape(1, H)
    b2r = b2.reshape(1, O)

    r2 = H // _NT2

    working = (4 * I * H
               + 2 * H * O
               + 4 * 2 * r2 * O
               + 2 * 2 * tm * H
               + 2 * 4 * (tm * I + tm * O)
               + 4 * (tm * r2 + H + O))
    vmem_limit = int(min(max(working + 12 * 1024 * 1024, 4 * 1024 * 1024),
                         58 * 1024 * 1024))

    cost = pl.CostEstimate(
        flops=2 * M * (I * H + H * O),
        transcendentals=0,
        bytes_accessed=4 * (M * I + I * H + H + H * O + O + M * O),
    )

    gm = grid_m

    out = pl.pallas_call(
        _make_mlp_kernel(grid_m, tm),
        out_shape=jax.ShapeDtypeStruct((M, O), x.dtype),
        grid=(grid_m + 1,),
        in_specs=[
            pl.BlockSpec((tm, I), lambda i: (jnp.minimum(i, gm - 1), 0)),
            pl.BlockSpec(memory_space=pl.ANY),
            pl.BlockSpec((1, H), lambda i: (0, 0)),
            pl.BlockSpec(memory_space=pl.ANY),
            pl.BlockSpec((1, O), lambda i: (0, 0)),
        ],
        out_specs=pl.BlockSpec((tm, O),
                               lambda i: (jnp.maximum(i - 1, 0), 0)),
        scratch_shapes=[
            pltpu.VMEM((I, H), jnp.float32),
            pltpu.VMEM((H, O), jnp.bfloat16),
            pltpu.VMEM((2, r2, O), jnp.float32),
            pltpu.VMEM((2 * tm, H), jnp.bfloat16),
            pltpu.SemaphoreType.DMA((_NT,)),
            pltpu.SemaphoreType.DMA((_NT2,)),
        ],
        compiler_params=pltpu.CompilerParams(
            dimension_semantics=("arbitrary",),
            vmem_limit_bytes=vmem_limit,
        ),
        cost_estimate=cost,
    )(x2, w1, b1r, w2, b2r)

    return out.reshape(*lead_shape, O)


# speedup vs baseline: 1.0173x; 1.0173x over previous
import jax
import jax.numpy as jnp
from jax.experimental import pallas as pl
from jax.experimental.pallas import tpu as pltpu

_NT = 4


def _mlp_kernel(x_ref, w1_hbm, b1_ref, w2_hbm, b2_ref, o_ref,
                w1s, w2s, sem1, sem2):
    I, H = w1s.shape
    O = w2s.shape[1]
    r1 = I // _NT
    r2 = H // _NT
    i = pl.program_id(0)

    def c1(t):
        return pltpu.make_async_copy(
            w1_hbm.at[pl.ds(t * r1, r1), :], w1s.at[pl.ds(t * r1, r1), :],
            sem1.at[t])

    def c2(t):
        return pltpu.make_async_copy(
            w2_hbm.at[pl.ds(t * r2, r2), :], w2s.at[pl.ds(t * r2, r2), :],
            sem2.at[t])

    @pl.when(i == 0)
    def _first_step():
        for t in range(_NT):
            c1(t).start()
        for t in range(_NT):
            c2(t).start()
        x = x_ref[...]
        h = b1_ref[...] * jnp.ones((x.shape[0], 1), jnp.float32)
        for t in range(_NT):
            c1(t).wait()
            h = h + jnp.dot(x[:, t * r1:(t + 1) * r1],
                            w1s[pl.ds(t * r1, r1), :],
                            preferred_element_type=jnp.float32)
        h = jnp.maximum(h, 0.0)
        acc = b2_ref[...] * jnp.ones((x.shape[0], 1), jnp.float32)
        for t in range(_NT):
            c2(t).wait()
            acc = acc + jnp.dot(h[:, t * r2:(t + 1) * r2],
                                w2s[pl.ds(t * r2, r2), :],
                                preferred_element_type=jnp.float32)
        o_ref[...] = acc.astype(o_ref.dtype)

    @pl.when(i > 0)
    def _steady_state():
        h = jnp.dot(x_ref[...], w1s[...], preferred_element_type=jnp.float32)
        h = jnp.maximum(h + b1_ref[...], 0.0)
        out = jnp.dot(h, w2s[...],
                      preferred_element_type=jnp.float32) + b2_ref[...]
        o_ref[...] = out.astype(o_ref.dtype)


def kernel(x, w1, b1, w2, b2):
    I = x.shape[-1]
    H = w1.shape[1]
    O = w2.shape[1]
    lead_shape = x.shape[:-1]

    x2 = x.reshape(-1, I)
    M = x2.shape[0]

    if M <= 512:
        tm = M
    else:
        tm = 512
    grid_m = pl.cdiv(M, tm)

    b1r = b1.reshape(1, H)
    b2r = b2.reshape(1, O)

    working = (4 * (I * H + H * O)
               + 2 * 4 * (tm * I + tm * O)
               + 4 * (tm * H + H + O))
    vmem_limit = int(min(max(working + 8 * 1024 * 1024, 4 * 1024 * 1024),
                         58 * 1024 * 1024))

    cost = pl.CostEstimate(
        flops=2 * M * (I * H + H * O),
        transcendentals=0,
        bytes_accessed=4 * (M * I + I * H + H + H * O + O + M * O),
    )

    out = pl.pallas_call(
        _mlp_kernel,
        out_shape=jax.ShapeDtypeStruct((M, O), x.dtype),
        grid=(grid_m,),
        in_specs=[
            pl.BlockSpec((tm, I), lambda i: (i, 0)),
            pl.BlockSpec(memory_space=pl.ANY),
            pl.BlockSpec((1, H), lambda i: (0, 0)),
            pl.BlockSpec(memory_space=pl.ANY),
            pl.BlockSpec((1, O), lambda i: (0, 0)),
        ],
        out_specs=pl.BlockSpec((tm, O), lambda i: (i, 0)),
        scratch_shapes=[
            pltpu.VMEM((I, H), jnp.float32),
            pltpu.VMEM((H, O), jnp.float32),
            pltpu.SemaphoreType.DMA((_NT,)),
            pltpu.SemaphoreType.DMA((_NT,)),
        ],
        compiler_params=pltpu.CompilerParams(
            dimension_semantics=("arbitrary",),
            vmem_limit_bytes=vmem_limit,
        ),
        cost_estimate=cost,
    )(x2, w1, b1r, w2, b2r)

    return out.reshape(*lead_shape, O)
